# Initial kernel scaffold; baseline (speedup 1.0000x reference)
import jax
import jax.numpy as jnp
from jax.experimental import pallas as pl


def kernel(obs, rnn_states, masks, ln0_g, ln0_b, w1, b1, ln1_g, ln1_b, w2, b2, ln2_g, ln2_b, wm, bm, log_std):
    raise NotImplementedError("write your pallas kernel here")


# speedup vs baseline: 1.3363x; 1.3363x over previous
import math

import jax
import jax.numpy as jnp
from jax.experimental import pallas as pl
from jax.experimental.pallas import tpu as pltpu

_OBS = 16
_H = 32
_ACT = 4
_EPS = 1e-5
_LOG_2PI = math.log(2.0 * math.pi)

_W_ROWS = _OBS + _H + _H
_V_ROWS = 8


def _actor_kernel(obs_ref, w_ref, v_ref, act_ref, lp_ref):
    v = v_ref[...]
    x = obs_ref[...]

    def stats(h, width):
        inv_n = 1.0 / width
        s1 = jnp.sum(h, axis=-1, keepdims=True)
        s2 = jnp.sum(h * h, axis=-1, keepdims=True)
        mu = s1 * inv_n
        var = s2 * inv_n - mu * mu
        return mu, jax.lax.rsqrt(var + _EPS)

    mu0, r0 = stats(x, _OBS)
    y1 = jnp.dot(x, w_ref[0:_OBS, :], preferred_element_type=jnp.float32)
    h1 = jnp.maximum((y1 - mu0 * v[1:2, :]) * r0 + v[0:1, :], 0.0)

    mu1, r1 = stats(h1, _H)
    y2 = jnp.dot(h1, w_ref[_OBS:_OBS + _H, :],
                 preferred_element_type=jnp.float32)
    h2 = jnp.maximum((y2 - mu1 * v[3:4, :]) * r1 + v[2:3, :], 0.0)

    mu2, r2 = stats(h2, _H)
    y3 = jnp.dot(h2, w_ref[_OBS + _H:, :], preferred_element_type=jnp.float32)
    mean = (y3 - mu2 * v[5:6, :]) * r2 + v[4:5, :]

    act_ref[...] = mean[:, :_ACT]
    lp_ref[...] = jnp.broadcast_to(v[6:7, 0:1], lp_ref.shape)


def kernel(obs, rnn_states, masks, ln0_g, ln0_b, w1, b1, ln1_g, ln1_b,
           w2, b2, ln2_g, ln2_b, wm, bm, log_std):
    del masks
    B = obs.shape[0]
    f32 = jnp.float32

    w1f = (w1 * ln0_g.T).astype(f32)
    c1 = (ln0_b @ w1 + b1).astype(f32)
    s1w = jnp.sum(w1f, axis=0, keepdims=True)

    w2f = (w2 * ln1_g.T).astype(f32)
    c2 = (ln1_b @ w2 + b2).astype(f32)
    s2w = jnp.sum(w2f, axis=0, keepdims=True)

    wmf = (wm * ln2_g.T).astype(f32)
    cm = (ln2_b @ wm + bm).astype(f32)
    smw = jnp.sum(wmf, axis=0, keepdims=True)

    def pad_cols(a):
        return jnp.pad(a, ((0, 0), (0, _H - a.shape[1])))

    w_slab = jnp.concatenate([w1f, w2f, pad_cols(wmf)], axis=0)

    lp = jnp.sum(-log_std - 0.5 * _LOG_2PI)
    lp_row = jnp.full((1, _H), lp, f32)
    v_slab = jnp.concatenate(
        [c1, s1w, c2, s2w, pad_cols(cm), pad_cols(smw), lp_row,
         jnp.zeros((1, _H), f32)], axis=0)

    tile_b = 4096 if (B % 4096 == 0) else B
    grid = (B // tile_b,)

    cost = pl.CostEstimate(
        flops=2 * B * (_OBS * _H + _H * _H + _H * _H),
        transcendentals=3 * B,
        bytes_accessed=4 * (B * _OBS + _W_ROWS * _H + _V_ROWS * _H
                            + B * _ACT + B),
    )

    actions, log_probs = pl.pallas_call(
        _actor_kernel,
        out_shape=[jax.ShapeDtypeStruct((B, _ACT), f32),
                   jax.ShapeDtypeStruct((B, 1), f32)],
        grid=grid,
        in_specs=[
            pl.BlockSpec((tile_b, _OBS), lambda i: (i, 0)),
            pl.BlockSpec((_W_ROWS, _H), lambda i: (0, 0)),
            pl.BlockSpec((_V_ROWS, _H), lambda i: (0, 0)),
        ],
        out_specs=[pl.BlockSpec((tile_b, _ACT), lambda i: (i, 0)),
                   pl.BlockSpec((tile_b, 1), lambda i: (i, 0))],
        compiler_params=pltpu.CompilerParams(
            dimension_semantics=("parallel",)),
        cost_estimate=cost,
    )(obs, w_slab, v_slab)

    return actions, log_probs, rnn_states


# speedup vs baseline: 2.0064x; 1.5015x over previous
import math

import jax
import jax.numpy as jnp
from jax.experimental import pallas as pl
from jax.experimental.pallas import tpu as pltpu

_OBS = 16
_H = 32
_ACT = 4
_EPS = 1e-5
_LOG_2PI = math.log(2.0 * math.pi)
_G = 8


def _actor_kernel(xp_ref, e16_ref, w1_ref, w2_ref, wm_ref, cb_ref, cm_ref,
                  act_ref, lp_ref):
    n = xp_ref.shape[0]
    xp = xp_ref[...]

    inv16 = 1.0 / _OBS
    s1 = jnp.dot(xp, e16_ref[...], preferred_element_type=jnp.float32)
    s2 = jnp.dot(xp * xp, e16_ref[...], preferred_element_type=jnp.float32)
    mu = s1 * inv16
    var = s2 * inv16 - mu * mu
    xn_p = (xp - mu) * jax.lax.rsqrt(var + _EPS)

    xn = xn_p.T

    def ln_stats(g, width):
        inv_n = 1.0 / width
        a = jnp.sum(g, axis=1, keepdims=True)
        b = jnp.sum(g * g, axis=1, keepdims=True)
        m = a * inv_n
        v = b * inv_n - m * m
        return m, jax.lax.rsqrt(v + _EPS)

    y1 = jnp.dot(w1_ref[...], xn, preferred_element_type=jnp.float32)
    h1 = jnp.maximum(y1 + cb_ref[:, 0:1], 0.0)

    g1 = h1.reshape(_G, _H, n)
    mu1, r1 = ln_stats(g1, _H)
    h1n = ((g1 - mu1) * r1).reshape(_G * _H, n)

    y2 = jnp.dot(w2_ref[...], h1n, preferred_element_type=jnp.float32)
    h2 = jnp.maximum(y2 + cb_ref[:, 1:2], 0.0)

    g2 = h2.reshape(_G, _H, n)
    mu2, r2 = ln_stats(g2, _H)

    y3 = jnp.dot(wm_ref[...], h2, preferred_element_type=jnp.float32)
    y3g = y3.reshape(_G, _ACT, n)
    t2 = mu2 * r2
    wms_g = cm_ref[:, 1:2].reshape(_G, _ACT, 1)
    cm_g = cm_ref[:, 0:1].reshape(_G, _ACT, 1)
    mean = (y3g * r2 - wms_g * t2 + cm_g).reshape(_G * _ACT, n)

    act_ref[...] = mean.T
    lp_ref[...] = jnp.broadcast_to(cm_ref[0:1, 2:3], lp_ref.shape)


def kernel(obs, rnn_states, masks, ln0_g, ln0_b, w1, b1, ln1_g, ln1_b,
           w2, b2, ln2_g, ln2_b, wm, bm, log_std):
    del masks
    B = obs.shape[0]
    f32 = jnp.float32

    w1f = (w1 * ln0_g.T).astype(f32)
    c1 = (ln0_b @ w1 + b1).astype(f32)
    w2f = (w2 * ln1_g.T).astype(f32)
    c2 = (ln1_b @ w2 + b2).astype(f32)
    wmf = (wm * ln2_g.T).astype(f32)
    cm = (ln2_b @ wm + bm).astype(f32)

    eye = jnp.eye(_G, dtype=f32)
    e16 = jnp.kron(eye, jnp.ones((_OBS, _OBS), f32))
    bdw1 = jnp.kron(eye, w1f.T)
    bdw2 = jnp.kron(eye, w2f.T)
    bdwm = jnp.kron(eye, wmf.T)

    ones_col = jnp.ones((_G, 1), f32)
    c1col = jnp.kron(ones_col, c1.T)
    c2col = jnp.kron(ones_col, c2.T)
    cmcol = jnp.kron(ones_col, cm.T)
    wms = jnp.sum(wmf, axis=0, keepdims=True)
    wmscol = jnp.kron(ones_col, wms.T)
    lp = jnp.sum(-log_std - 0.5 * _LOG_2PI)
    cb = jnp.concatenate([c1col, c2col], axis=1)
    cmx = jnp.concatenate(
        [cmcol, wmscol, jnp.full((_G * _ACT, 1), lp, f32)], axis=1)

    P = B // _G
    xp = obs.reshape(P, _G * _OBS)

    tile = 4096 if P % 4096 == 0 else P
    grid = (P // tile,)

    mm = tile * (256 * 128 + 256 * 256 + 32 * 256 + 2 * 128 * 128)
    cost = pl.CostEstimate(
        flops=2 * (P // tile) * mm,
        transcendentals=3 * B,
        bytes_accessed=4 * (B * _OBS + B * _ACT + B),
    )

    act_p, lp_p = pl.pallas_call(
        _actor_kernel,
        out_shape=[jax.ShapeDtypeStruct((P, _G * _ACT), f32),
                   jax.ShapeDtypeStruct((P * _G // 128, 128), f32)],
        grid=grid,
        in_specs=[
            pl.BlockSpec((tile, _G * _OBS), lambda i: (i, 0)),
            pl.BlockSpec(e16.shape, lambda i: (0, 0)),
            pl.BlockSpec(bdw1.shape, lambda i: (0, 0)),
            pl.BlockSpec(bdw2.shape, lambda i: (0, 0)),
            pl.BlockSpec(bdwm.shape, lambda i: (0, 0)),
            pl.BlockSpec(cb.shape, lambda i: (0, 0)),
            pl.BlockSpec(cmx.shape, lambda i: (0, 0)),
        ],
        out_specs=[pl.BlockSpec((tile, _G * _ACT), lambda i: (i, 0)),
                   pl.BlockSpec((tile * _G // 128, 128), lambda i: (i, 0))],
        compiler_params=pltpu.CompilerParams(
            dimension_semantics=("parallel",)),
        cost_estimate=cost,
    )(xp, e16, bdw1, bdw2, bdwm, cb, cmx)

    actions = act_p.reshape(B, _ACT)
    log_probs = lp_p.reshape(B, 1)
    return actions, log_probs, rnn_states
